# baseline (speedup 1.0000x reference)
import jax
import jax.numpy as jnp
from jax.experimental import pallas as pl
from jax.experimental.pallas import tpu as pltpu
from jax.experimental.pallas import tpu_sc as plsc

_BBLK = 8
_NCHUNK = 2


def _sc_gather(idx, table):
    nb, seq = idx.shape
    _, embed = table.shape

    mesh = plsc.VectorSubcoreMesh(
        core_axis_name="core", subcore_axis_name="subcore"
    )

    @pl.kernel(
        out_type=jax.ShapeDtypeStruct((nb, seq, embed), table.dtype),
        mesh=mesh,
        scratch_types=[pltpu.SemaphoreType.DMA],
    )
    def _gather(tab_hbm, idx_hbm, out_hbm, sem):
        def body(i_vmem, o_vmem):
            copies = [
                pltpu.async_copy(
                    tab_hbm.at[i_vmem.at[j]], o_vmem.at[j], sem
                )
                for j in range(_BBLK)
            ]
            for c in copies:
                c.wait()

        pltpu.emit_pipeline(
            body,
            grid=(nb // _BBLK,),
            in_specs=[
                pl.BlockSpec((_BBLK, seq), index_map=lambda i: (i, 0))
            ],
            out_specs=[
                pl.BlockSpec(
                    (_BBLK, seq, embed), index_map=lambda i: (i, 0, 0)
                )
            ],
            core_axis_name=("core", "subcore"),
            dimension_semantics=(pltpu.PARALLEL,),
        )(idx_hbm, out_hbm)

    return _gather(table, idx)


def kernel(word_ids, table):
    batch, _ = word_ids.shape
    idx = word_ids.astype(jnp.int32)
    chunk = batch // _NCHUNK
    outs = [
        _sc_gather(idx[k * chunk : (k + 1) * chunk], table)
        for k in range(_NCHUNK)
    ]
    return jnp.concatenate(outs, axis=0)
